# Initial kernel scaffold; baseline (speedup 1.0000x reference)
import jax
import jax.numpy as jnp
from jax.experimental import pallas as pl


def kernel(indices, weight):
    raise NotImplementedError("write your pallas kernel here")


# speedup vs baseline: 2.4240x; 2.4240x over previous
import functools

import jax
import jax.numpy as jnp
from jax import lax
from jax.experimental import pallas as pl
from jax.experimental.pallas import tpu as pltpu
from jax.experimental.pallas import tpu_sc as plsc

DATASET = 100000
D = 128
BATCH = 4096
HIST = 50
N = BATCH * HIST

NC = 2
NS = 16
NW = NC * NS
B_PER_W = N // NW
CHUNK = 128
NCHUNK = B_PER_W // CHUNK
LANES = 16


def _emb_body(idx_hbm, table_hbm, out_hbm, idx_v, rows_v, gsem):
    wid = lax.axis_index("s") * NC + lax.axis_index("c")
    base = wid * B_PER_W

    pltpu.sync_copy(idx_hbm.at[wid], idx_v)

    def chunk_body(g, carry):
        buf = rows_v.at[0]
        pltpu.async_copy(table_hbm.at[idx_v.at[g]], buf, gsem).wait()

        def relu_row(r, c):
            for j in range(D // LANES):
                sl = pl.ds(j * LANES, LANES)
                buf[r, sl] = jnp.maximum(buf[r, sl], 0.0)
            return c

        lax.fori_loop(0, CHUNK, relu_row, 0)

        pltpu.sync_copy(buf, out_hbm.at[pl.ds(base + g * CHUNK, CHUNK)])
        return carry

    lax.fori_loop(0, NCHUNK, chunk_body, 0)


@functools.partial(jax.jit, static_argnums=())
def _emb_call(idx3, weight):
    mesh = plsc.VectorSubcoreMesh(core_axis_name="c", subcore_axis_name="s")
    fn = functools.partial(
        pl.kernel,
        mesh=mesh,
        out_type=jax.ShapeDtypeStruct((N, D), jnp.float32),
        scratch_types=[
            pltpu.VMEM((NCHUNK, CHUNK), jnp.int32),
            pltpu.VMEM((1, CHUNK, D), jnp.float32),
            pltpu.SemaphoreType.DMA,
        ],
    )(_emb_body)
    return fn(idx3, weight)


def kernel(indices, weight):
    idx3 = indices.reshape(NW, NCHUNK, CHUNK)
    out = _emb_call(idx3, weight)
    return out.reshape(BATCH, HIST, D)


# speedup vs baseline: 2.9603x; 1.2213x over previous
import functools

import jax
import jax.numpy as jnp
from jax import lax
from jax.experimental import pallas as pl
from jax.experimental.pallas import tpu as pltpu
from jax.experimental.pallas import tpu_sc as plsc

DATASET = 100000
D = 128
BATCH = 4096
HIST = 50
N = BATCH * HIST

NC = 2
NS = 16
NW = NC * NS
B_PER_W = N // NW
CHUNK = 128
NCHUNK = B_PER_W // CHUNK
NBUF = 5
LANES = 16


def _emb_body(idx_hbm, table_hbm, out_hbm, idx_v, rows_v, *sems):
    gsems = sems[:NBUF]
    ssems = sems[NBUF:]
    wid = lax.axis_index("s") * NC + lax.axis_index("c")
    base = wid * B_PER_W

    pltpu.sync_copy(idx_hbm.at[wid], idx_v)

    def gather(g, b):
        return pltpu.async_copy(table_hbm.at[idx_v.at[g]], rows_v.at[b],
                                gsems[b])

    for b in range(NBUF):
        gather(b, b)

    def outer(i, carry):
        for b in range(NBUF):
            g = i * NBUF + b
            buf = rows_v.at[b]
            pltpu.make_async_copy(table_hbm.at[idx_v.at[g]], buf,
                                  gsems[b]).wait()

            def relu_row(r, c):
                for j in range(D // LANES):
                    sl = pl.ds(j * LANES, LANES)
                    buf[r, sl] = jnp.maximum(buf[r, sl], 0.0)
                return c

            lax.fori_loop(0, CHUNK, relu_row, 0)

            pltpu.async_copy(buf, out_hbm.at[pl.ds(base + g * CHUNK, CHUNK)],
                             ssems[b])

            @pl.when(g + NBUF < NCHUNK)
            def _():
                pltpu.make_async_copy(
                    buf, out_hbm.at[pl.ds(base + g * CHUNK, CHUNK)],
                    ssems[b]).wait()
                gather(g + NBUF, b)

        return carry

    lax.fori_loop(0, NCHUNK // NBUF, outer, 0)

    for b in range(NBUF):
        g = NCHUNK - NBUF + b
        pltpu.make_async_copy(rows_v.at[b],
                              out_hbm.at[pl.ds(base + g * CHUNK, CHUNK)],
                              ssems[b]).wait()


def _emb_call(idx3, weight):
    mesh = plsc.VectorSubcoreMesh(core_axis_name="c", subcore_axis_name="s")
    fn = functools.partial(
        pl.kernel,
        mesh=mesh,
        out_type=jax.ShapeDtypeStruct((N, D), jnp.float32),
        scratch_types=[
            pltpu.VMEM((NCHUNK, CHUNK), jnp.int32),
            pltpu.VMEM((NBUF, CHUNK, D), jnp.float32),
        ] + [pltpu.SemaphoreType.DMA] * (2 * NBUF),
    )(_emb_body)
    return fn(idx3, weight)


def kernel(indices, weight):
    idx3 = indices.reshape(NW, NCHUNK, CHUNK)
    out = _emb_call(idx3, weight)
    return out.reshape(BATCH, HIST, D)
